# baseline (speedup 1.0000x reference)
import functools

import jax
import jax.numpy as jnp
from jax import lax
from jax.experimental import pallas as pl
from jax.experimental.pallas import tpu as pltpu
from jax.experimental.pallas import tpu_sc as plsc

N = 16384
K = 8192
D = 256

RB = 4096
CB = 512
NR = N // RB
NCB = K // CB

W_BOUNDS = (0, 2736, 5472, 8192)

SC_CORES = 2
SC_SUBCORES = 16
NW = SC_CORES * SC_SUBCORES
BPW = N // NW
CH = 128
NCH = BPW // CH


def _dist_argmin_body(e2_ref, x_ref, em_ref, idx_ref):
    x = x_ref[...]
    x2 = jnp.sum(x * x, axis=1)
    pos_inf = jnp.float32(jnp.inf)
    bigf = jnp.float32(2 * K)
    jf = lax.broadcasted_iota(jnp.int32, (RB, CB), 1).astype(jnp.float32)

    ms = [None, None, None]
    is_ = [None, None, None]
    for c in range(NCB):
        b0, b1 = c * CB, (c + 1) * CB
        em = em_ref[pl.ds(b0, CB), :]
        e2 = e2_ref[pl.ds(b0, CB)]
        dotn = lax.dot_general(x, em, (((1,), (1,)), ((), ())),
                               preferred_element_type=jnp.float32)
        t = (x2[:, None] + dotn) + e2[None, :]
        for w in range(3):
            lo, hi = W_BOUNDS[w], W_BOUNDS[w + 1]
            if hi <= b0 or lo >= b1:
                continue
            if lo <= b0 and hi >= b1:
                tw = t
            else:
                mask = (jf >= jnp.float32(lo - b0)) & (jf < jnp.float32(hi - b0))
                tw = jnp.where(mask, t, pos_inf)
            lmin = jnp.min(tw, axis=1)
            larg = (jnp.min(jnp.where(tw == lmin[:, None], jf, bigf), axis=1)
                    + jnp.float32(b0))
            if ms[w] is None:
                ms[w], is_[w] = lmin, larg
            else:
                better = lmin < ms[w]
                ms[w] = jnp.where(better, lmin, ms[w])
                is_[w] = jnp.where(better, larg, is_[w])

    m = ms[0].astype(jnp.bfloat16).astype(jnp.float32)
    i = is_[0]
    b1_ = ms[1] < m
    m = jnp.where(b1_, ms[1], m)
    i = jnp.where(b1_, is_[1], i)
    m = m.astype(jnp.bfloat16).astype(jnp.float32)
    b2_ = ms[2] < m
    i = jnp.where(b2_, is_[2], i)
    idx_ref[...] = i.astype(jnp.int32)


_dist_argmin = pl.pallas_call(
    _dist_argmin_body,
    grid=(NR,),
    in_specs=[
        pl.BlockSpec((K,), lambda r: (0,)),
        pl.BlockSpec((RB, D), lambda r: (r, 0)),
        pl.BlockSpec((K, D), lambda r: (0, 0)),
    ],
    out_specs=pl.BlockSpec((RB,), lambda r: (r,)),
    out_shape=jax.ShapeDtypeStruct((N,), jnp.int32),
    compiler_params=pltpu.CompilerParams(
        dimension_semantics=("arbitrary",)),
)


def _gather_body(idx_hbm, tab_hbm, out_hbm, idx_v, buf_a, buf_b, sem_a, sem_b):
    wid = lax.axis_index("s") * SC_CORES + lax.axis_index("c")
    base = wid * BPW
    pltpu.sync_copy(idx_hbm.at[wid], idx_v)
    bufs = (buf_a, buf_b)
    sems = (sem_a, sem_b)
    copies = []
    for j in range(NCH):
        b = j % 2
        if j >= 2:
            copies[j - 2].wait()
            pltpu.sync_copy(bufs[b], out_hbm.at[pl.ds(base + (j - 2) * CH, CH)])
        copies.append(
            pltpu.async_copy(tab_hbm.at[idx_v.at[j]], bufs[b], sems[b]))
    for j in range(NCH - 2, NCH):
        copies[j].wait()
        pltpu.sync_copy(bufs[j % 2], out_hbm.at[pl.ds(base + j * CH, CH)])


@functools.cache
def _make_gather():
    return functools.partial(
        pl.kernel,
        mesh=plsc.VectorSubcoreMesh(core_axis_name="c", subcore_axis_name="s"),
        out_type=jax.ShapeDtypeStruct((N, D), jnp.float32),
        scratch_types=[
            pltpu.VMEM((NCH, CH), jnp.int32),
            pltpu.VMEM((CH, D), jnp.float32),
            pltpu.VMEM((CH, D), jnp.float32),
            pltpu.SemaphoreType.DMA,
            pltpu.SemaphoreType.DMA,
        ],
    )(_gather_body)


def kernel(input_tensor, emb_weights):
    flat = input_tensor.reshape(N, D)
    e2 = jnp.sum(emb_weights.T ** 2, axis=0)
    em = emb_weights * jnp.float32(-2.0)
    idx = _dist_argmin(e2, flat, em)
    quantized = _make_gather()(idx.reshape(NW, NCH, CH), emb_weights)
    return quantized.reshape(input_tensor.shape)


# speedup vs baseline: 1.2488x; 1.2488x over previous
import functools

import jax
import jax.numpy as jnp
from jax import lax
from jax.experimental import pallas as pl
from jax.experimental.pallas import tpu as pltpu
from jax.experimental.pallas import tpu_sc as plsc

N = 16384
K = 8192
D = 256

RB = 2048
CB = 512
NR = N // RB
NCB = K // CB

W_BOUNDS = (0, 2736, 5472, 8192)

SC_CORES = 2
SC_SUBCORES = 16
NW = SC_CORES * SC_SUBCORES
BPW = N // NW
CH = 128
NCH = BPW // CH


def _dist_argmin_body(e2_ref, x_ref, em_ref, idx_ref):
    x = x_ref[...]
    x2 = jnp.sum(x * x, axis=1)
    pos_inf = jnp.float32(jnp.inf)
    bigf = jnp.float32(2 * K)
    jf = lax.broadcasted_iota(jnp.int32, (RB, CB), 1).astype(jnp.float32)

    ms = [None, None, None]
    is_ = [None, None, None]
    for c in range(NCB):
        b0, b1 = c * CB, (c + 1) * CB
        em = em_ref[pl.ds(b0, CB), :] * jnp.float32(-2.0)
        e2 = e2_ref[pl.ds(b0, CB)]
        dotn = lax.dot_general(x, em, (((1,), (1,)), ((), ())),
                               preferred_element_type=jnp.float32)
        t = (x2[:, None] + dotn) + e2[None, :]
        for w in range(3):
            lo, hi = W_BOUNDS[w], W_BOUNDS[w + 1]
            if hi <= b0 or lo >= b1:
                continue
            if lo <= b0 and hi >= b1:
                tw = t
            else:
                mask = (jf >= jnp.float32(lo - b0)) & (jf < jnp.float32(hi - b0))
                tw = jnp.where(mask, t, pos_inf)
            lmin = jnp.min(tw, axis=1)
            larg = (jnp.min(jnp.where(tw == lmin[:, None], jf, bigf), axis=1)
                    + jnp.float32(b0))
            if ms[w] is None:
                ms[w], is_[w] = lmin, larg
            else:
                better = lmin < ms[w]
                ms[w] = jnp.where(better, lmin, ms[w])
                is_[w] = jnp.where(better, larg, is_[w])

    m = ms[0].astype(jnp.bfloat16).astype(jnp.float32)
    i = is_[0]
    b1_ = ms[1] < m
    m = jnp.where(b1_, ms[1], m)
    i = jnp.where(b1_, is_[1], i)
    m = m.astype(jnp.bfloat16).astype(jnp.float32)
    b2_ = ms[2] < m
    i = jnp.where(b2_, is_[2], i)
    idx_ref[...] = i.astype(jnp.int32)


_dist_argmin = pl.pallas_call(
    _dist_argmin_body,
    grid=(NR,),
    in_specs=[
        pl.BlockSpec((K,), lambda r: (0,)),
        pl.BlockSpec((RB, D), lambda r: (r, 0)),
        pl.BlockSpec((K, D), lambda r: (0, 0)),
    ],
    out_specs=pl.BlockSpec((RB,), lambda r: (r,)),
    out_shape=jax.ShapeDtypeStruct((N,), jnp.int32),
    compiler_params=pltpu.CompilerParams(
        dimension_semantics=("arbitrary",)),
)


def _gather_body(idx_hbm, tab_hbm, out_hbm, idx_v, buf_a, buf_b, sem_a, sem_b):
    wid = lax.axis_index("s") * SC_CORES + lax.axis_index("c")
    base = wid * BPW
    pltpu.sync_copy(idx_hbm.at[wid], idx_v)
    bufs = (buf_a, buf_b)
    sems = (sem_a, sem_b)
    copies = []
    for j in range(NCH):
        b = j % 2
        if j >= 2:
            copies[j - 2].wait()
            pltpu.sync_copy(bufs[b], out_hbm.at[pl.ds(base + (j - 2) * CH, CH)])
        copies.append(
            pltpu.async_copy(tab_hbm.at[idx_v.at[j]], bufs[b], sems[b]))
    for j in range(NCH - 2, NCH):
        copies[j].wait()
        pltpu.sync_copy(bufs[j % 2], out_hbm.at[pl.ds(base + j * CH, CH)])


@functools.cache
def _make_gather():
    return functools.partial(
        pl.kernel,
        mesh=plsc.VectorSubcoreMesh(core_axis_name="c", subcore_axis_name="s"),
        out_type=jax.ShapeDtypeStruct((N, D), jnp.float32),
        scratch_types=[
            pltpu.VMEM((NCH, CH), jnp.int32),
            pltpu.VMEM((CH, D), jnp.float32),
            pltpu.VMEM((CH, D), jnp.float32),
            pltpu.SemaphoreType.DMA,
            pltpu.SemaphoreType.DMA,
        ],
    )(_gather_body)


def kernel(input_tensor, emb_weights):
    flat = input_tensor.reshape(N, D)
    e2 = jnp.sum(emb_weights.T ** 2, axis=0)
    idx = _dist_argmin(e2, flat, emb_weights)
    quantized = _make_gather()(idx.reshape(NW, NCH, CH), emb_weights)
    return quantized.reshape(input_tensor.shape)
